# baseline (speedup 1.0000x reference)
import functools

import jax
import jax.numpy as jnp
from jax import lax
from jax.experimental import pallas as pl
from jax.experimental.pallas import tpu as pltpu
from jax.experimental.pallas import tpu_sc as plsc

DIM = 16
L = 16
NC, NS = 2, 16
NW = NC * NS
GRP = 8


def _sc_scores(u, i, j, W, H):
    B = u.shape[0]
    bpw = B // NW

    u2 = u.reshape(NW, bpw)
    i2 = i.reshape(NW, bpw)
    j2 = j.reshape(NW, bpw)
    WT = jnp.transpose(W)
    HT = jnp.transpose(H)

    mesh = plsc.VectorSubcoreMesh(
        core_axis_name="c", subcore_axis_name="s",
        num_cores=NC, num_subcores=NS)

    @functools.partial(
        pl.kernel,
        out_type=jax.ShapeDtypeStruct((NW, bpw), jnp.float32),
        mesh=mesh,
        scratch_types=[
            pltpu.VMEM((bpw + L,), jnp.int32),
            pltpu.VMEM((bpw + L,), jnp.int32),
            pltpu.VMEM((bpw + L,), jnp.int32),
            pltpu.VMEM((GRP * DIM, 128), jnp.float32),
            pltpu.VMEM((GRP * DIM, 128), jnp.float32),
            pltpu.VMEM((GRP * DIM, 128), jnp.float32),
            pltpu.VMEM((GRP * DIM, 128), jnp.float32),
            pltpu.VMEM((GRP * DIM, 128), jnp.float32),
            pltpu.VMEM((GRP * DIM, 128), jnp.float32),
            pltpu.VMEM((bpw + L,), jnp.float32),
            pltpu.SemaphoreType.DMA,
            pltpu.SemaphoreType.DMA,
        ],
        compiler_params=pltpu.CompilerParams(
            use_tc_tiling_on_sc=True, needs_layout_passes=False),
    )
    def sc(u_hbm, i_hbm, j_hbm, wt_hbm, ht_hbm, out_hbm,
           u_v, i_v, j_v, ws0, his0, hjs0, ws1, his1, hjs1, x_v, sem0, sem1):
        wid = lax.axis_index("s") * NC + lax.axis_index("c")
        pltpu.sync_copy(u_hbm.at[wid], u_v.at[pl.ds(0, bpw)])
        pltpu.sync_copy(i_hbm.at[wid], i_v.at[pl.ds(0, bpw)])
        pltpu.sync_copy(j_hbm.at[wid], j_v.at[pl.ds(0, bpw)])

        lane = lax.iota(jnp.int32, L)
        ngrp = bpw // GRP

        def fetch(g, ws, his, hjs, sem):
            sl = pl.ds(g * GRP, L)
            ub = (u_v[sl] >> 7) * 128
            ib = (i_v[sl] >> 7) * 128
            jb = (j_v[sl] >> 7) * 128
            for t in range(GRP):
                dst = pl.ds(t * DIM, DIM)
                ru = pl.multiple_of(ub[t], 128)
                ri = pl.multiple_of(ib[t], 128)
                rj = pl.multiple_of(jb[t], 128)
                pltpu.async_copy(wt_hbm.at[:, pl.ds(ru, 128)], ws.at[dst], sem)
                pltpu.async_copy(ht_hbm.at[:, pl.ds(ri, 128)], his.at[dst], sem)
                pltpu.async_copy(ht_hbm.at[:, pl.ds(rj, 128)], hjs.at[dst], sem)

        def drain(ws, his, hjs, sem):
            src = wt_hbm.at[:, pl.ds(0, 128)]
            for t in range(GRP):
                dst = pl.ds(t * DIM, DIM)
                pltpu.make_async_copy(src, ws.at[dst], sem).wait()
                pltpu.make_async_copy(src, his.at[dst], sem).wait()
                pltpu.make_async_copy(src, hjs.at[dst], sem).wait()

        def compute(g, ws, his, hjs):
            sl = pl.ds(g * GRP, L)
            cu = u_v[sl] & 127
            ci = i_v[sl] & 127
            cj = j_v[sl] & 127
            rowbase = (lane & (GRP - 1)) * DIM
            acc = jnp.zeros((L,), jnp.float32)
            for d in range(DIM):
                rf = rowbase + d
                wu = plsc.load_gather(ws, [rf, cu])
                hi = plsc.load_gather(his, [rf, ci])
                hj = plsc.load_gather(hjs, [rf, cj])
                acc = acc + wu * (hi - hj)
            plsc.store_compressed(x_v.at[sl], acc, mask=lane < GRP)

        fetch(0, ws0, his0, hjs0, sem0)
        fetch(1, ws1, his1, hjs1, sem1)

        def body(k, carry):
            drain(ws0, his0, hjs0, sem0)
            compute(2 * k, ws0, his0, hjs0)

            @pl.when(k < ngrp // 2 - 1)
            def _():
                fetch(2 * k + 2, ws0, his0, hjs0, sem0)

            drain(ws1, his1, hjs1, sem1)
            compute(2 * k + 1, ws1, his1, hjs1)

            @pl.when(k < ngrp // 2 - 1)
            def _():
                fetch(2 * k + 3, ws1, his1, hjs1, sem1)

            return carry

        lax.fori_loop(0, ngrp // 2, body, 0)
        pltpu.sync_copy(x_v.at[pl.ds(0, bpw)], out_hbm.at[wid])

    return sc(u2, i2, j2, WT, HT).reshape(B)


def _tc_loss(x):
    B = x.shape[0]
    xm = x.reshape(B // 128, 128)

    def body(x_ref, o_ref):
        t = -x_ref[...]
        sp = jnp.maximum(t, 0.0) + jnp.log1p(jnp.exp(-jnp.abs(t)))
        o_ref[0, 0] = jnp.sum(sp) * (1.0 / B)

    out = pl.pallas_call(
        body,
        out_shape=jax.ShapeDtypeStruct((1, 1), jnp.float32),
        out_specs=pl.BlockSpec(memory_space=pltpu.SMEM),
    )(xm)
    return out[0, 0]


def kernel(u, i, j, W, H):
    x = _sc_scores(u, i, j, W, H)
    return _tc_loss(x)


# speedup vs baseline: 1.0404x; 1.0404x over previous
import functools

import jax
import jax.numpy as jnp
from jax import lax
from jax.experimental import pallas as pl
from jax.experimental.pallas import tpu as pltpu
from jax.experimental.pallas import tpu_sc as plsc

DIM = 16
L = 16
NC, NS = 2, 16
NW = NC * NS
GRP = 16


def _sc_scores(u, i, j, W, H):
    B = u.shape[0]
    bpw = B // NW

    u2 = u.reshape(NW, bpw)
    i2 = i.reshape(NW, bpw)
    j2 = j.reshape(NW, bpw)
    WT = jnp.transpose(W)
    HT = jnp.transpose(H)

    mesh = plsc.VectorSubcoreMesh(
        core_axis_name="c", subcore_axis_name="s",
        num_cores=NC, num_subcores=NS)

    @functools.partial(
        pl.kernel,
        out_type=jax.ShapeDtypeStruct((NW, bpw), jnp.float32),
        mesh=mesh,
        scratch_types=[
            pltpu.VMEM((bpw,), jnp.int32),
            pltpu.VMEM((bpw,), jnp.int32),
            pltpu.VMEM((bpw,), jnp.int32),
            pltpu.VMEM((GRP * DIM, 128), jnp.float32),
            pltpu.VMEM((GRP * DIM, 128), jnp.float32),
            pltpu.VMEM((GRP * DIM, 128), jnp.float32),
            pltpu.VMEM((bpw,), jnp.float32),
            pltpu.SemaphoreType.DMA,
        ],
        compiler_params=pltpu.CompilerParams(
            use_tc_tiling_on_sc=True, needs_layout_passes=False),
    )
    def sc(u_hbm, i_hbm, j_hbm, wt_hbm, ht_hbm, out_hbm,
           u_v, i_v, j_v, ws_v, his_v, hjs_v, x_v, sem):
        wid = lax.axis_index("s") * NC + lax.axis_index("c")
        pltpu.sync_copy(u_hbm.at[wid], u_v)
        pltpu.sync_copy(i_hbm.at[wid], i_v)
        pltpu.sync_copy(j_hbm.at[wid], j_v)

        lane = lax.iota(jnp.int32, L)

        def fetch(g, carry):
            sl = pl.ds(g * GRP, GRP)
            uvec = u_v[sl]
            ivec = i_v[sl]
            jvec = j_v[sl]
            ub = (uvec >> 7) * 128
            ib = (ivec >> 7) * 128
            jb = (jvec >> 7) * 128
            copies = []
            for t in range(GRP):
                dst = pl.ds(t * DIM, DIM)
                ru = pl.multiple_of(ub[t], 128)
                ri = pl.multiple_of(ib[t], 128)
                rj = pl.multiple_of(jb[t], 128)
                copies.append(pltpu.async_copy(
                    wt_hbm.at[:, pl.ds(ru, 128)], ws_v.at[dst], sem))
                copies.append(pltpu.async_copy(
                    ht_hbm.at[:, pl.ds(ri, 128)], his_v.at[dst], sem))
                copies.append(pltpu.async_copy(
                    ht_hbm.at[:, pl.ds(rj, 128)], hjs_v.at[dst], sem))
            for cp in copies:
                cp.wait()

            cu = uvec & 127
            ci = ivec & 127
            cj = jvec & 127
            rowbase = lane * DIM
            acc = jnp.zeros((L,), jnp.float32)
            for d in range(DIM):
                rf = rowbase + d
                wu = plsc.load_gather(ws_v, [rf, cu])
                hi = plsc.load_gather(his_v, [rf, ci])
                hj = plsc.load_gather(hjs_v, [rf, cj])
                acc = acc + wu * (hi - hj)
            x_v[sl] = acc
            return carry

        lax.fori_loop(0, bpw // GRP, fetch, 0)
        pltpu.sync_copy(x_v, out_hbm.at[wid])

    return sc(u2, i2, j2, WT, HT).reshape(B)


def _tc_loss(x):
    B = x.shape[0]
    xm = x.reshape(B // 128, 128)

    def body(x_ref, o_ref):
        t = -x_ref[...]
        sp = jnp.maximum(t, 0.0) + jnp.log1p(jnp.exp(-jnp.abs(t)))
        o_ref[0, 0] = jnp.sum(sp) * (1.0 / B)

    out = pl.pallas_call(
        body,
        out_shape=jax.ShapeDtypeStruct((1, 1), jnp.float32),
        out_specs=pl.BlockSpec(memory_space=pltpu.SMEM),
    )(xm)
    return out[0, 0]


def kernel(u, i, j, W, H):
    x = _sc_scores(u, i, j, W, H)
    return _tc_loss(x)
